# Initial kernel scaffold; baseline (speedup 1.0000x reference)
import functools

import jax
import jax.numpy as jnp
from jax import lax
from jax.experimental import pallas as pl
from jax.experimental.pallas import tpu as pltpu
from jax.experimental.pallas import tpu_sc as plsc

_N_NODES = 10000
_N_EDGES = 320000
_D = 128

_NW = 32
_ROWS_PER_W = 320
_N_PAD = _NW * _ROWS_PER_W
_CH = 3200
_NCHUNK = _N_EDGES // _CH
_NGRP = _CH // 16
_GW = 128

_SCORE_W = 128


def _mesh():
    return plsc.VectorSubcoreMesh(core_axis_name="core", subcore_axis_name="subcore")


def _pool_body(x_ref, w_ref, b_ref, o_ref):
    acc = jnp.dot(x_ref[...], w_ref[...], precision=lax.Precision.HIGHEST,
                  preferred_element_type=jnp.float32)
    o_ref[...] = jnp.maximum(acc + b_ref[...], 0.0)


def _pool_tc(x, W_pool, b_pool):
    n, d = x.shape
    blk = 1250
    return pl.pallas_call(
        _pool_body,
        grid=(n // blk,),
        in_specs=[
            pl.BlockSpec((blk, d), lambda i: (i, 0)),
            pl.BlockSpec((d, d), lambda i: (0, 0)),
            pl.BlockSpec((1, d), lambda i: (0, 0)),
        ],
        out_specs=pl.BlockSpec((blk, d), lambda i: (i, 0)),
        out_shape=jax.ShapeDtypeStruct((n, d), jnp.float32),
    )(x, W_pool, b_pool.reshape(1, d))


def _combine_body(x_ref, n_ref, ws_ref, wn_ref, b_ref, o_ref):
    a = jnp.dot(x_ref[...], ws_ref[...], precision=lax.Precision.HIGHEST,
                preferred_element_type=jnp.float32)
    c = jnp.dot(n_ref[...], wn_ref[...], precision=lax.Precision.HIGHEST,
                preferred_element_type=jnp.float32)
    o_ref[...] = a + c + b_ref[...]


def _combine_tc(x, neigh, W_self, W_neigh, b):
    n, d = x.shape
    blk = 1250
    return pl.pallas_call(
        _combine_body,
        grid=(n // blk,),
        in_specs=[
            pl.BlockSpec((blk, d), lambda i: (i, 0)),
            pl.BlockSpec((blk, d), lambda i: (i, 0)),
            pl.BlockSpec((d, d), lambda i: (0, 0)),
            pl.BlockSpec((d, d), lambda i: (0, 0)),
            pl.BlockSpec((1, d), lambda i: (0, 0)),
        ],
        out_specs=pl.BlockSpec((blk, d), lambda i: (i, 0)),
        out_shape=jax.ShapeDtypeStruct((n, d), jnp.float32),
    )(x, neigh, W_self, W_neigh, b.reshape(1, d))


def _segmax_body(hpool_hbm, src_hbm, dst_hbm, out_hbm,
                 acc, dstbuf, srcbuf, mdst, msrc, msgbuf):
    wid = lax.axis_index("core") * 16 + lax.axis_index("subcore")
    lo = wid * _ROWS_PER_W
    hi = lo + _ROWS_PER_W
    iota = lax.broadcasted_iota(jnp.int32, (16,), 0)
    zf16 = jnp.zeros((16,), jnp.float32)
    zi16 = jnp.zeros((16,), jnp.int32)

    @pl.loop(0, _ROWS_PER_W)
    def _(r):
        for jv in range(_D // 16):
            acc[r, pl.ds(jv * 16, 16)] = zf16

    @pl.loop(0, (_CH + _GW) // 16)
    def _(i):
        plsc.store_scatter(msrc, [i * 16 + iota], zi16)

    def do_chunk(c, _):
        pltpu.sync_copy(dst_hbm.at[c], dstbuf)
        pltpu.sync_copy(src_hbm.at[c], srcbuf)

        def filt(g, moff):
            dvec = dstbuf[g, :]
            svec = srcbuf[g, :]
            mask = (dvec >= lo) & (dvec < hi)
            plsc.store_compressed(mdst.at[pl.ds(moff, 16)], dvec - lo, mask=mask)
            plsc.store_compressed(msrc.at[pl.ds(moff, 16)], svec, mask=mask)
            return moff + jnp.sum(mask.astype(jnp.int32))

        mo = lax.fori_loop(0, _NGRP, filt, jnp.int32(0))

        nsub = (mo + _GW - 1) // _GW

        def do_sub(s, _):
            base = s * _GW
            pltpu.sync_copy(hpool_hbm.at[msrc.at[pl.ds(base, _GW)]], msgbuf)
            cnt = jnp.minimum(_GW, mo - base)

            def rmw(p, _):
                psplat = jnp.full((16,), p, jnp.int32)
                dsplat = plsc.load_gather(mdst, [jnp.full((16,), base + p, jnp.int32)])
                for jv in range(_D // 16):
                    col = iota + jv * 16
                    m = plsc.load_gather(msgbuf, [psplat, col])
                    a = plsc.load_gather(acc, [dsplat, col])
                    plsc.store_scatter(acc, [dsplat, col], jnp.maximum(a, m))
                return 0

            lax.fori_loop(0, cnt, rmw, 0)
            return 0

        lax.fori_loop(0, nsub, do_sub, 0)
        return 0

    lax.fori_loop(0, _NCHUNK, do_chunk, 0)

    pltpu.sync_copy(acc, out_hbm.at[pl.ds(lo, _ROWS_PER_W)])


def _segmax_sc(h_pool, src, dst):
    dst3 = dst.reshape(_NCHUNK, _NGRP, 16)
    src3 = src.reshape(_NCHUNK, _NGRP, 16)
    kfn = pl.kernel(
        _segmax_body,
        out_type=jax.ShapeDtypeStruct((_N_PAD, _D), jnp.float32),
        mesh=_mesh(),
        scratch_types=[
            pltpu.VMEM((_ROWS_PER_W, _D), jnp.float32),
            pltpu.VMEM((_NGRP, 16), jnp.int32),
            pltpu.VMEM((_NGRP, 16), jnp.int32),
            pltpu.VMEM((_CH + _GW,), jnp.int32),
            pltpu.VMEM((_CH + _GW,), jnp.int32),
            pltpu.VMEM((_GW, _D), jnp.float32),
        ],
    )
    return kfn(h_pool, src3, dst3)


def _score_body(h_hbm, src_hbm, dst_hbm, out_hbm, sbuf):
    def body(s_ref, d_ref, o_ref):
        pltpu.sync_copy(h_hbm.at[s_ref.at[0]], sbuf)
        pltpu.sync_copy(h_hbm.at[d_ref.at[0]], o_ref)

        @pl.loop(0, _SCORE_W)
        def _(r):
            for jv in range(_D // 16):
                sl = pl.ds(jv * 16, 16)
                o_ref[r, sl] = o_ref[r, sl] * sbuf[r, sl]

    pltpu.emit_pipeline(
        body,
        grid=(_N_EDGES // _SCORE_W,),
        in_specs=[
            pl.BlockSpec((1, _SCORE_W), lambda i: (0, i)),
            pl.BlockSpec((1, _SCORE_W), lambda i: (0, i)),
        ],
        out_specs=[pl.BlockSpec((_SCORE_W, _D), lambda i: (i, 0))],
        core_axis_name=("core", "subcore"),
        dimension_semantics=(pltpu.PARALLEL,),
    )(src_hbm, dst_hbm, out_hbm)


def _score_sc(h, src, dst):
    kfn = pl.kernel(
        _score_body,
        out_type=jax.ShapeDtypeStruct((_N_EDGES, _D), jnp.float32),
        mesh=_mesh(),
        scratch_types=[pltpu.VMEM((_SCORE_W, _D), jnp.float32)],
    )
    return kfn(h, src.reshape(1, _N_EDGES), dst.reshape(1, _N_EDGES))


def kernel(x, edge_index, W_pool, b_pool, W_self, W_neigh, b):
    src = edge_index[0].astype(jnp.int32)
    dst = edge_index[1].astype(jnp.int32)
    h_pool = _pool_tc(x, W_pool, b_pool)
    neigh = _segmax_sc(h_pool, src, dst)[:_N_NODES]
    h = _combine_tc(x, neigh, W_self, W_neigh, b)
    return _score_sc(h, src, dst)


# speedup vs baseline: 1.5239x; 1.5239x over previous
import dataclasses
import functools

import jax
import jax.numpy as jnp
from jax import lax
from jax.experimental import pallas as pl
from jax.experimental.pallas import tpu as pltpu
from jax.experimental.pallas import tpu_sc as plsc

_N_NODES = 10000
_N_EDGES = 320000
_D = 128

_NW = 32
_ROWS_PER_W = 320
_N_PAD = _NW * _ROWS_PER_W
_CH = 3200
_NCHUNK = _N_EDGES // _CH
_NGRP = _CH // 16
_GW = 128

_SCORE_W = 128


def _mesh():
    return plsc.VectorSubcoreMesh(core_axis_name="core", subcore_axis_name="subcore")


def _sc_params():
    cp = pltpu.CompilerParams()
    if "needs_layout_passes" in pltpu.CompilerParams.__dataclass_fields__:
        cp = dataclasses.replace(cp, needs_layout_passes=False)
    return cp


def _pool_body(x_ref, w_ref, b_ref, o_ref):
    acc = jnp.dot(x_ref[...], w_ref[...], precision=lax.Precision.HIGHEST,
                  preferred_element_type=jnp.float32)
    o_ref[...] = jnp.maximum(acc + b_ref[...], 0.0)


def _pool_tc(x, W_pool, b_pool):
    n, d = x.shape
    blk = 2000
    return pl.pallas_call(
        _pool_body,
        grid=(n // blk,),
        in_specs=[
            pl.BlockSpec((blk, d), lambda i: (i, 0)),
            pl.BlockSpec((d, d), lambda i: (0, 0)),
            pl.BlockSpec((1, d), lambda i: (0, 0)),
        ],
        out_specs=pl.BlockSpec((blk, d), lambda i: (i, 0)),
        out_shape=jax.ShapeDtypeStruct((n, d), jnp.float32),
    )(x, W_pool, b_pool.reshape(1, d))


def _combine_body(x_ref, n_ref, ws_ref, wn_ref, b_ref, o_ref):
    a = jnp.dot(x_ref[...], ws_ref[...], precision=lax.Precision.HIGHEST,
                preferred_element_type=jnp.float32)
    c = jnp.dot(n_ref[...], wn_ref[...], precision=lax.Precision.HIGHEST,
                preferred_element_type=jnp.float32)
    o_ref[...] = a + c + b_ref[...]


def _combine_tc(x, neigh, W_self, W_neigh, b):
    n, d = x.shape
    blk = 2000
    return pl.pallas_call(
        _combine_body,
        grid=(n // blk,),
        in_specs=[
            pl.BlockSpec((blk, d), lambda i: (i, 0)),
            pl.BlockSpec((blk, d), lambda i: (i, 0)),
            pl.BlockSpec((d, d), lambda i: (0, 0)),
            pl.BlockSpec((d, d), lambda i: (0, 0)),
            pl.BlockSpec((1, d), lambda i: (0, 0)),
        ],
        out_specs=pl.BlockSpec((blk, d), lambda i: (i, 0)),
        out_shape=jax.ShapeDtypeStruct((n, d), jnp.float32),
    )(x, neigh, W_self, W_neigh, b.reshape(1, d))


def _segmax_body(hpool_hbm, src_hbm, dst_hbm, out_hbm,
                 acc, dstbuf, srcbuf, mdst, msrc, msgbuf):
    wid = lax.axis_index("core") * 16 + lax.axis_index("subcore")
    lo = wid * _ROWS_PER_W
    hi = lo + _ROWS_PER_W
    iota = lax.broadcasted_iota(jnp.int32, (16,), 0)
    zf16 = jnp.zeros((16,), jnp.float32)
    zi16 = jnp.zeros((16,), jnp.int32)

    @pl.loop(0, _ROWS_PER_W)
    def _(r):
        for jv in range(_D // 16):
            acc[r, pl.ds(jv * 16, 16)] = zf16

    @pl.loop(0, (_CH + _GW) // 16)
    def _(i):
        plsc.store_scatter(msrc, [i * 16 + iota], zi16)

    def do_chunk(c, _):
        pltpu.sync_copy(dst_hbm.at[c], dstbuf)
        pltpu.sync_copy(src_hbm.at[c], srcbuf)

        def filt(g, moff):
            dvec = dstbuf[g, :]
            svec = srcbuf[g, :]
            mask = (dvec >= lo) & (dvec < hi)
            plsc.store_compressed(mdst.at[pl.ds(moff, 16)], dvec - lo, mask=mask)
            plsc.store_compressed(msrc.at[pl.ds(moff, 16)], svec, mask=mask)
            return moff + jnp.sum(mask.astype(jnp.int32))

        mo = lax.fori_loop(0, _NGRP, filt, jnp.int32(0))

        nsub = (mo + _GW - 1) // _GW

        def do_sub(s, _):
            base = s * _GW
            pltpu.sync_copy(hpool_hbm.at[msrc.at[pl.ds(base, _GW)]], msgbuf)
            cnt = jnp.minimum(_GW, mo - base)

            def rmw(p, _):
                psplat = jnp.full((16,), p, jnp.int32)
                dsplat = plsc.load_gather(mdst, [jnp.full((16,), base + p, jnp.int32)])
                for jv in range(_D // 16):
                    col = iota + jv * 16
                    m = plsc.load_gather(msgbuf, [psplat, col])
                    a = plsc.load_gather(acc, [dsplat, col])
                    plsc.store_scatter(acc, [dsplat, col], jnp.maximum(a, m))
                return 0

            lax.fori_loop(0, cnt, rmw, 0)
            return 0

        lax.fori_loop(0, nsub, do_sub, 0)
        return 0

    lax.fori_loop(0, _NCHUNK, do_chunk, 0)

    pltpu.sync_copy(acc, out_hbm.at[pl.ds(lo, _ROWS_PER_W)])


def _segmax_sc(h_pool, src, dst):
    dst3 = dst.reshape(_NCHUNK, _NGRP, 16)
    src3 = src.reshape(_NCHUNK, _NGRP, 16)
    kfn = pl.kernel(
        _segmax_body,
        out_type=jax.ShapeDtypeStruct((_N_PAD, _D), jnp.float32),
        mesh=_mesh(),
        scratch_types=[
            pltpu.VMEM((_ROWS_PER_W, _D), jnp.float32),
            pltpu.VMEM((_NGRP, 16), jnp.int32),
            pltpu.VMEM((_NGRP, 16), jnp.int32),
            pltpu.VMEM((_CH + _GW,), jnp.int32),
            pltpu.VMEM((_CH + _GW,), jnp.int32),
            pltpu.VMEM((_GW, _D), jnp.float32),
        ],
        compiler_params=_sc_params(),
    )
    return kfn(h_pool, src3, dst3)


def _score_body(h_hbm, src_hbm, dst_hbm, out_hbm, sbuf):
    def body(s_ref, d_ref, o_ref):
        pltpu.sync_copy(h_hbm.at[s_ref.at[0]], sbuf)
        pltpu.sync_copy(h_hbm.at[d_ref.at[0]], o_ref)

        @pl.loop(0, _SCORE_W)
        def _(r):
            for jv in range(_D // 16):
                sl = pl.ds(jv * 16, 16)
                o_ref[r, sl] = o_ref[r, sl] * sbuf[r, sl]

    pltpu.emit_pipeline(
        body,
        grid=(_N_EDGES // _SCORE_W,),
        in_specs=[
            pl.BlockSpec((1, _SCORE_W), lambda i: (0, i)),
            pl.BlockSpec((1, _SCORE_W), lambda i: (0, i)),
        ],
        out_specs=[pl.BlockSpec((_SCORE_W, _D), lambda i: (i, 0))],
        core_axis_name=("core", "subcore"),
        dimension_semantics=(pltpu.PARALLEL,),
    )(src_hbm, dst_hbm, out_hbm)


def _score_sc(h, src, dst):
    kfn = pl.kernel(
        _score_body,
        out_type=jax.ShapeDtypeStruct((_N_EDGES, _D), jnp.float32),
        mesh=_mesh(),
        scratch_types=[pltpu.VMEM((_SCORE_W, _D), jnp.float32)],
        compiler_params=_sc_params(),
    )
    return kfn(h, src.reshape(1, _N_EDGES), dst.reshape(1, _N_EDGES))


def kernel(x, edge_index, W_pool, b_pool, W_self, W_neigh, b):
    src = edge_index[0].astype(jnp.int32)
    dst = edge_index[1].astype(jnp.int32)
    h_pool = _pool_tc(x, W_pool, b_pool)
    neigh = _segmax_sc(h_pool, src, dst)[:_N_NODES]
    h = _combine_tc(x, neigh, W_self, W_neigh, b)
    return _score_sc(h, src, dst)


# speedup vs baseline: 1.7292x; 1.1347x over previous
import dataclasses
import functools

import jax
import jax.numpy as jnp
from jax import lax
from jax.experimental import pallas as pl
from jax.experimental.pallas import tpu as pltpu
from jax.experimental.pallas import tpu_sc as plsc

_N_NODES = 10000
_N_EDGES = 320000
_D = 128

_NW = 32
_ROWS_PER_W = 320
_N_PAD = _NW * _ROWS_PER_W
_CH = 3200
_NCHUNK = _N_EDGES // _CH
_NGRP = _CH // 16
_CROWS = _CH // 128
_GW = 128

_SCORE_W = 128


def _mesh():
    return plsc.VectorSubcoreMesh(core_axis_name="core", subcore_axis_name="subcore")


def _sc_params():
    cp = pltpu.CompilerParams()
    if "needs_layout_passes" in pltpu.CompilerParams.__dataclass_fields__:
        cp = dataclasses.replace(cp, needs_layout_passes=False)
    return cp


def _pool_body(x_ref, w_ref, b_ref, o_ref):
    acc = jnp.dot(x_ref[...], w_ref[...], precision=lax.Precision.HIGHEST,
                  preferred_element_type=jnp.float32)
    o_ref[...] = jnp.maximum(acc + b_ref[...], 0.0)


def _pool_tc(x, W_pool, b_pool):
    n, d = x.shape
    blk = 2000
    return pl.pallas_call(
        _pool_body,
        grid=(n // blk,),
        in_specs=[
            pl.BlockSpec((blk, d), lambda i: (i, 0)),
            pl.BlockSpec((d, d), lambda i: (0, 0)),
            pl.BlockSpec((1, d), lambda i: (0, 0)),
        ],
        out_specs=pl.BlockSpec((blk, d), lambda i: (i, 0)),
        out_shape=jax.ShapeDtypeStruct((n, d), jnp.float32),
    )(x, W_pool, b_pool.reshape(1, d))


def _combine_body(x_ref, n_ref, ws_ref, wn_ref, b_ref, o_ref):
    a = jnp.dot(x_ref[...], ws_ref[...], precision=lax.Precision.HIGHEST,
                preferred_element_type=jnp.float32)
    c = jnp.dot(n_ref[...], wn_ref[...], precision=lax.Precision.HIGHEST,
                preferred_element_type=jnp.float32)
    o_ref[...] = a + c + b_ref[...]


def _combine_tc(x, neigh, W_self, W_neigh, b):
    n, d = x.shape
    blk = 2000
    return pl.pallas_call(
        _combine_body,
        grid=(n // blk,),
        in_specs=[
            pl.BlockSpec((blk, d), lambda i: (i, 0)),
            pl.BlockSpec((blk, d), lambda i: (i, 0)),
            pl.BlockSpec((d, d), lambda i: (0, 0)),
            pl.BlockSpec((d, d), lambda i: (0, 0)),
            pl.BlockSpec((1, d), lambda i: (0, 0)),
        ],
        out_specs=pl.BlockSpec((blk, d), lambda i: (i, 0)),
        out_shape=jax.ShapeDtypeStruct((n, d), jnp.float32),
    )(x, neigh, W_self, W_neigh, b.reshape(1, d))


def _segmax_body(hpool_hbm, src_hbm, dst_hbm, out_hbm,
                 acc, dstbuf, srcbuf, mdst, msrc, msgbuf, ssem, gsem):
    wid = lax.axis_index("core") * 16 + lax.axis_index("subcore")
    lo = wid * _ROWS_PER_W
    hi = lo + _ROWS_PER_W
    iota = lax.broadcasted_iota(jnp.int32, (16,), 0)
    zf16 = jnp.zeros((16,), jnp.float32)
    zi16 = jnp.zeros((16,), jnp.int32)

    @pl.loop(0, _ROWS_PER_W)
    def _(r):
        for jv in range(_D // 16):
            acc[r, pl.ds(jv * 16, 16)] = zf16

    @pl.loop(0, (_CH + _GW) // 16)
    def _(i):
        plsc.store_scatter(msrc, [i * 16 + iota], zi16)

    def issue_scan(c, k):
        pltpu.async_copy(dst_hbm.at[c], dstbuf.at[pl.ds(k * _CROWS, _CROWS)],
                         ssem.at[k])
        pltpu.async_copy(src_hbm.at[c], srcbuf.at[pl.ds(k * _CROWS, _CROWS)],
                         ssem.at[k])

    def wait_scan(c, k):
        pltpu.make_async_copy(dst_hbm.at[c], dstbuf.at[pl.ds(k * _CROWS, _CROWS)],
                              ssem.at[k]).wait()
        pltpu.make_async_copy(src_hbm.at[c], srcbuf.at[pl.ds(k * _CROWS, _CROWS)],
                              ssem.at[k]).wait()

    def issue_gather(s, j):
        pltpu.async_copy(hpool_hbm.at[msrc.at[pl.ds(s * _GW, _GW)]],
                         msgbuf.at[pl.ds(j * _GW, _GW)], gsem.at[j])

    def wait_gather(s, j):
        pltpu.make_async_copy(hpool_hbm.at[msrc.at[pl.ds(s * _GW, _GW)]],
                              msgbuf.at[pl.ds(j * _GW, _GW)], gsem.at[j]).wait()

    issue_scan(0, 0)

    def do_chunk(c, _):
        k = lax.rem(c, 2)

        @pl.when(c + 1 < _NCHUNK)
        def _():
            issue_scan(c + 1, 1 - k)

        wait_scan(c, k)

        def filt(g, moff):
            row = k * _CROWS + g
            for jj in range(128 // 16):
                dvec = dstbuf[row, pl.ds(jj * 16, 16)]
                svec = srcbuf[row, pl.ds(jj * 16, 16)]
                mask = (dvec >= lo) & (dvec < hi)
                plsc.store_compressed(mdst.at[pl.ds(moff, 16)], dvec - lo,
                                      mask=mask)
                plsc.store_compressed(msrc.at[pl.ds(moff, 16)], svec, mask=mask)
                moff = moff + jnp.sum(mask.astype(jnp.int32))
            return moff

        mo = lax.fori_loop(0, _CROWS, filt, jnp.int32(0))

        nsub = (mo + _GW - 1) // _GW

        @pl.when(nsub > 0)
        def _():
            issue_gather(0, 0)

        def do_sub(s, _):
            j = lax.rem(s, 2)

            @pl.when(s + 1 < nsub)
            def _():
                issue_gather(s + 1, 1 - j)

            wait_gather(s, j)
            base = s * _GW
            cnt = jnp.minimum(_GW, mo - base)

            def rmw(p, _):
                mrow = j * _GW + p
                dsplat = plsc.load_gather(mdst, [jnp.full((16,), base + p, jnp.int32)])
                for jv in range(_D // 16):
                    col = iota + jv * 16
                    m = msgbuf[mrow, pl.ds(jv * 16, 16)]
                    a = plsc.load_gather(acc, [dsplat, col])
                    plsc.store_scatter(acc, [dsplat, col], jnp.maximum(a, m))
                return 0

            lax.fori_loop(0, cnt, rmw, 0)
            return 0

        lax.fori_loop(0, nsub, do_sub, 0)
        return 0

    lax.fori_loop(0, _NCHUNK, do_chunk, 0)

    pltpu.sync_copy(acc, out_hbm.at[pl.ds(lo, _ROWS_PER_W)])


def _segmax_sc(h_pool, src, dst):
    dst3 = dst.reshape(_NCHUNK, _CROWS, 128)
    src3 = src.reshape(_NCHUNK, _CROWS, 128)
    kfn = pl.kernel(
        _segmax_body,
        out_type=jax.ShapeDtypeStruct((_N_PAD, _D), jnp.float32),
        mesh=_mesh(),
        scratch_types=[
            pltpu.VMEM((_ROWS_PER_W, _D), jnp.float32),
            pltpu.VMEM((2 * _CROWS, 128), jnp.int32),
            pltpu.VMEM((2 * _CROWS, 128), jnp.int32),
            pltpu.VMEM((_CH + _GW,), jnp.int32),
            pltpu.VMEM((_CH + _GW,), jnp.int32),
            pltpu.VMEM((2 * _GW, _D), jnp.float32),
            pltpu.SemaphoreType.DMA((2,)),
            pltpu.SemaphoreType.DMA((2,)),
        ],
        compiler_params=_sc_params(),
    )
    return kfn(h_pool, src3, dst3)


def _score_body(h_hbm, src_hbm, dst_hbm, out_hbm, sbuf):
    def body(s_ref, d_ref, o_ref):
        pltpu.sync_copy(h_hbm.at[s_ref.at[0]], sbuf)
        pltpu.sync_copy(h_hbm.at[d_ref.at[0]], o_ref)

        @pl.loop(0, _SCORE_W)
        def _(r):
            for jv in range(_D // 16):
                sl = pl.ds(jv * 16, 16)
                o_ref[r, sl] = o_ref[r, sl] * sbuf[r, sl]

    pltpu.emit_pipeline(
        body,
        grid=(_N_EDGES // _SCORE_W,),
        in_specs=[
            pl.BlockSpec((1, _SCORE_W), lambda i: (0, i)),
            pl.BlockSpec((1, _SCORE_W), lambda i: (0, i)),
        ],
        out_specs=[pl.BlockSpec((_SCORE_W, _D), lambda i: (i, 0))],
        core_axis_name=("core", "subcore"),
        dimension_semantics=(pltpu.PARALLEL,),
    )(src_hbm, dst_hbm, out_hbm)


def _score_sc(h, src, dst):
    kfn = pl.kernel(
        _score_body,
        out_type=jax.ShapeDtypeStruct((_N_EDGES, _D), jnp.float32),
        mesh=_mesh(),
        scratch_types=[pltpu.VMEM((_SCORE_W, _D), jnp.float32)],
        compiler_params=_sc_params(),
    )
    return kfn(h, src.reshape(1, _N_EDGES), dst.reshape(1, _N_EDGES))


def kernel(x, edge_index, W_pool, b_pool, W_self, W_neigh, b):
    src = edge_index[0].astype(jnp.int32)
    dst = edge_index[1].astype(jnp.int32)
    h_pool = _pool_tc(x, W_pool, b_pool)
    neigh = _segmax_sc(h_pool, src, dst)[:_N_NODES]
    h = _combine_tc(x, neigh, W_self, W_neigh, b)
    return _score_sc(h, src, dst)


# speedup vs baseline: 1.7962x; 1.0387x over previous
import dataclasses
import functools

import jax
import jax.numpy as jnp
from jax import lax
from jax.experimental import pallas as pl
from jax.experimental.pallas import tpu as pltpu
from jax.experimental.pallas import tpu_sc as plsc

_N_NODES = 10000
_N_EDGES = 320000
_D = 128

_NW = 32
_ROWS_PER_W = 320
_N_PAD = _NW * _ROWS_PER_W
_CH = 3200
_NCHUNK = _N_EDGES // _CH
_NGRP = _CH // 16
_CROWS = _CH // 128
_GW = 128

_SCORE_W = 128


def _mesh():
    return plsc.VectorSubcoreMesh(core_axis_name="core", subcore_axis_name="subcore")


def _sc_params():
    cp = pltpu.CompilerParams()
    if "needs_layout_passes" in pltpu.CompilerParams.__dataclass_fields__:
        cp = dataclasses.replace(cp, needs_layout_passes=False)
    return cp


def _pool_body(x_ref, w_ref, b_ref, o_ref):
    acc = jnp.dot(x_ref[...], w_ref[...], precision=lax.Precision.HIGHEST,
                  preferred_element_type=jnp.float32)
    o_ref[...] = jnp.maximum(acc + b_ref[...], 0.0)


def _pool_tc(x, W_pool, b_pool):
    n, d = x.shape
    blk = 2000
    return pl.pallas_call(
        _pool_body,
        grid=(n // blk,),
        in_specs=[
            pl.BlockSpec((blk, d), lambda i: (i, 0)),
            pl.BlockSpec((d, d), lambda i: (0, 0)),
            pl.BlockSpec((1, d), lambda i: (0, 0)),
        ],
        out_specs=pl.BlockSpec((blk, d), lambda i: (i, 0)),
        out_shape=jax.ShapeDtypeStruct((n, d), jnp.float32),
    )(x, W_pool, b_pool.reshape(1, d))


def _combine_body(x_ref, n_ref, ws_ref, wn_ref, b_ref, o_ref):
    a = jnp.dot(x_ref[...], ws_ref[...], precision=lax.Precision.HIGHEST,
                preferred_element_type=jnp.float32)
    c = jnp.dot(n_ref[...], wn_ref[...], precision=lax.Precision.HIGHEST,
                preferred_element_type=jnp.float32)
    o_ref[...] = a + c + b_ref[...]


def _combine_tc(x, neigh, W_self, W_neigh, b):
    n, d = x.shape
    blk = 2000
    return pl.pallas_call(
        _combine_body,
        grid=(n // blk,),
        in_specs=[
            pl.BlockSpec((blk, d), lambda i: (i, 0)),
            pl.BlockSpec((blk, d), lambda i: (i, 0)),
            pl.BlockSpec((d, d), lambda i: (0, 0)),
            pl.BlockSpec((d, d), lambda i: (0, 0)),
            pl.BlockSpec((1, d), lambda i: (0, 0)),
        ],
        out_specs=pl.BlockSpec((blk, d), lambda i: (i, 0)),
        out_shape=jax.ShapeDtypeStruct((n, d), jnp.float32),
    )(x, neigh, W_self, W_neigh, b.reshape(1, d))


def _segmax_body(hpool_hbm, src_hbm, dst_hbm, out_hbm,
                 acc, dstbuf, srcbuf, mdst, msrc, msgbuf, ssem, gsem):
    wid = lax.axis_index("core") * 16 + lax.axis_index("subcore")
    lo = wid * _ROWS_PER_W
    hi = lo + _ROWS_PER_W
    iota = lax.broadcasted_iota(jnp.int32, (16,), 0)
    zf16 = jnp.zeros((16,), jnp.float32)
    zi16 = jnp.zeros((16,), jnp.int32)

    @pl.loop(0, _ROWS_PER_W)
    def _(r):
        for jv in range(_D // 16):
            acc[r, pl.ds(jv * 16, 16)] = zf16

    @pl.loop(0, (_CH + _GW) // 16)
    def _(i):
        plsc.store_scatter(msrc, [i * 16 + iota], zi16)

    def issue_scan(c, k):
        pltpu.async_copy(dst_hbm.at[c], dstbuf.at[pl.ds(k * _CROWS, _CROWS)],
                         ssem.at[k])
        pltpu.async_copy(src_hbm.at[c], srcbuf.at[pl.ds(k * _CROWS, _CROWS)],
                         ssem.at[k])

    def wait_scan(c, k):
        pltpu.make_async_copy(dst_hbm.at[c], dstbuf.at[pl.ds(k * _CROWS, _CROWS)],
                              ssem.at[k]).wait()
        pltpu.make_async_copy(src_hbm.at[c], srcbuf.at[pl.ds(k * _CROWS, _CROWS)],
                              ssem.at[k]).wait()

    def issue_gather(s, j):
        pltpu.async_copy(hpool_hbm.at[msrc.at[pl.ds(s * _GW, _GW)]],
                         msgbuf.at[pl.ds(j * _GW, _GW)], gsem.at[j])

    def wait_gather(s, j):
        pltpu.make_async_copy(hpool_hbm.at[msrc.at[pl.ds(s * _GW, _GW)]],
                              msgbuf.at[pl.ds(j * _GW, _GW)], gsem.at[j]).wait()

    issue_scan(0, 0)

    def do_chunk(c, _):
        k = lax.rem(c, 2)

        @pl.when(c + 1 < _NCHUNK)
        def _():
            issue_scan(c + 1, 1 - k)

        wait_scan(c, k)

        def filt(g, moff):
            row = k * _CROWS + g
            for jj in range(128 // 16):
                dvec = dstbuf[row, pl.ds(jj * 16, 16)]
                svec = srcbuf[row, pl.ds(jj * 16, 16)]
                mask = (dvec >= lo) & (dvec < hi)
                plsc.store_compressed(mdst.at[pl.ds(moff, 16)], dvec - lo,
                                      mask=mask)
                plsc.store_compressed(msrc.at[pl.ds(moff, 16)], svec, mask=mask)
                moff = moff + jnp.sum(mask.astype(jnp.int32))
            return moff

        mo = lax.fori_loop(0, _CROWS, filt, jnp.int32(0))

        nsub = (mo + _GW - 1) // _GW

        @pl.when(nsub > 0)
        def _():
            issue_gather(0, 0)

        def do_sub(s, _):
            j = lax.rem(s, 2)

            @pl.when(s + 1 < nsub)
            def _():
                issue_gather(s + 1, 1 - j)

            wait_gather(s, j)
            base = s * _GW
            cnt = jnp.minimum(_GW, mo - base)

            def rmw(p, _):
                mrow = j * _GW + p
                dsplat = plsc.load_gather(mdst, [jnp.full((16,), base + p, jnp.int32)])
                for jv in range(_D // 16):
                    col = iota + jv * 16
                    m = msgbuf[mrow, pl.ds(jv * 16, 16)]
                    a = plsc.load_gather(acc, [dsplat, col])
                    plsc.store_scatter(acc, [dsplat, col], jnp.maximum(a, m))
                return 0

            lax.fori_loop(0, cnt, rmw, 0)
            return 0

        lax.fori_loop(0, nsub, do_sub, 0)
        return 0

    lax.fori_loop(0, _NCHUNK, do_chunk, 0)

    pltpu.sync_copy(acc, out_hbm.at[pl.ds(lo, _ROWS_PER_W)])


def _segmax_sc(h_pool, src, dst):
    dst3 = dst.reshape(_NCHUNK, _CROWS, 128)
    src3 = src.reshape(_NCHUNK, _CROWS, 128)
    kfn = pl.kernel(
        _segmax_body,
        out_type=jax.ShapeDtypeStruct((_N_PAD, _D), jnp.float32),
        mesh=_mesh(),
        scratch_types=[
            pltpu.VMEM((_ROWS_PER_W, _D), jnp.float32),
            pltpu.VMEM((2 * _CROWS, 128), jnp.int32),
            pltpu.VMEM((2 * _CROWS, 128), jnp.int32),
            pltpu.VMEM((_CH + _GW,), jnp.int32),
            pltpu.VMEM((_CH + _GW,), jnp.int32),
            pltpu.VMEM((2 * _GW, _D), jnp.float32),
            pltpu.SemaphoreType.DMA((2,)),
            pltpu.SemaphoreType.DMA((2,)),
        ],
        compiler_params=_sc_params(),
    )
    return kfn(h_pool, src3, dst3)


def _score_body(h_hbm, src_hbm, dst_hbm, out_hbm, sbuf, gsem):
    def body(s_ref, d_ref, o_ref):
        pltpu.async_copy(h_hbm.at[s_ref.at[0]], sbuf, gsem.at[0])
        pltpu.async_copy(h_hbm.at[d_ref.at[0]], o_ref, gsem.at[1])
        pltpu.make_async_copy(h_hbm.at[s_ref.at[0]], sbuf, gsem.at[0]).wait()
        pltpu.make_async_copy(h_hbm.at[d_ref.at[0]], o_ref, gsem.at[1]).wait()

        @pl.loop(0, _SCORE_W)
        def _(r):
            for jv in range(_D // 16):
                sl = pl.ds(jv * 16, 16)
                o_ref[r, sl] = o_ref[r, sl] * sbuf[r, sl]

    pltpu.emit_pipeline(
        body,
        grid=(_N_EDGES // _SCORE_W,),
        in_specs=[
            pl.BlockSpec((1, _SCORE_W), lambda i: (0, i)),
            pl.BlockSpec((1, _SCORE_W), lambda i: (0, i)),
        ],
        out_specs=[pl.BlockSpec((_SCORE_W, _D), lambda i: (i, 0))],
        core_axis_name=("core", "subcore"),
        dimension_semantics=(pltpu.PARALLEL,),
    )(src_hbm, dst_hbm, out_hbm)


def _score_sc(h, src, dst):
    kfn = pl.kernel(
        _score_body,
        out_type=jax.ShapeDtypeStruct((_N_EDGES, _D), jnp.float32),
        mesh=_mesh(),
        scratch_types=[pltpu.VMEM((_SCORE_W, _D), jnp.float32),
                       pltpu.SemaphoreType.DMA((2,))],
        compiler_params=_sc_params(),
    )
    return kfn(h, src.reshape(1, _N_EDGES), dst.reshape(1, _N_EDGES))


def kernel(x, edge_index, W_pool, b_pool, W_self, W_neigh, b):
    src = edge_index[0].astype(jnp.int32)
    dst = edge_index[1].astype(jnp.int32)
    h_pool = _pool_tc(x, W_pool, b_pool)
    neigh = _segmax_sc(h_pool, src, dst)[:_N_NODES]
    h = _combine_tc(x, neigh, W_self, W_neigh, b)
    return _score_sc(h, src, dst)
